# baseline (device time: 46309 ns/iter reference)
import jax
import jax.numpy as jnp
from jax import lax
from jax.experimental import pallas as pl
from jax.experimental.pallas import tpu as pltpu

N_DEV = 16
N_TOK = 512
D_IN = 256
D_OUT = 512
N_EXP = 32
CHUNK = N_TOK // N_DEV


def kernel(x, router_W, route_idx, expert_W):
    def body(x_ref, rw_ref, idx_ref, ew_ref, out_ref,
             p_ref, send_buf, recv_buf, send_sems, recv_sems):
        my = lax.axis_index("i")
        left = lax.rem(my + N_DEV - 1, N_DEV)
        right = lax.rem(my + 1, N_DEV)

        barrier_sem = pltpu.get_barrier_semaphore()
        for nbr in (left, right):
            pl.semaphore_signal(
                barrier_sem, inc=1,
                device_id=(nbr,), device_id_type=pl.DeviceIdType.MESH,
            )
        pl.semaphore_wait(barrier_sem, 2)

        xv = x_ref[:, :]
        scores = jnp.dot(xv, rw_ref[:, :], preferred_element_type=jnp.float32)
        s_max = jnp.max(scores, axis=-1, keepdims=True)
        pexp = jnp.exp(scores - s_max)
        probs = pexp / jnp.sum(pexp, axis=-1, keepdims=True)

        e_ids = lax.broadcasted_iota(jnp.int32, (N_TOK, N_EXP), 1)
        top_mask = (e_ids == idx_ref[:, 0:1]) | (e_ids == idx_ref[:, 1:2])
        gp = jnp.where(top_mask, probs, 0.0)
        gates = gp / jnp.sum(gp, axis=-1, keepdims=True)

        g0 = jnp.sum(jnp.where(e_ids == 2 * my, gates, 0.0),
                     axis=-1, keepdims=True)
        g1 = jnp.sum(jnp.where(e_ids == 2 * my + 1, gates, 0.0),
                     axis=-1, keepdims=True)

        p_ref[:, :] = (
            g0 * jnp.dot(xv, ew_ref[0], preferred_element_type=jnp.float32)
            + g1 * jnp.dot(xv, ew_ref[1], preferred_element_type=jnp.float32)
        )

        c0 = lax.rem(my + N_DEV - 1, N_DEV)
        send_buf[0] = p_ref[pl.ds(c0 * CHUNK, CHUNK), :]
        for s in range(N_DEV - 1):
            rdma = pltpu.make_async_remote_copy(
                src_ref=send_buf.at[s],
                dst_ref=recv_buf.at[s],
                send_sem=send_sems.at[s],
                recv_sem=recv_sems.at[s],
                device_id=(right,),
                device_id_type=pl.DeviceIdType.MESH,
            )
            rdma.start()
            rdma.wait()
            c = lax.rem(my + N_DEV - 2 - s, N_DEV)
            chunk_sum = recv_buf[s] + p_ref[pl.ds(c * CHUNK, CHUNK), :]
            if s < N_DEV - 2:
                send_buf[s + 1] = chunk_sum
            else:
                out_ref[:, :] = chunk_sum

    return pl.pallas_call(
        body,
        out_shape=jax.ShapeDtypeStruct((CHUNK, D_OUT), jnp.float32),
        in_specs=[
            pl.BlockSpec(memory_space=pltpu.VMEM),
            pl.BlockSpec(memory_space=pltpu.VMEM),
            pl.BlockSpec(memory_space=pltpu.VMEM),
            pl.BlockSpec(memory_space=pltpu.VMEM),
        ],
        out_specs=pl.BlockSpec(memory_space=pltpu.VMEM),
        scratch_shapes=[
            pltpu.VMEM((N_TOK, D_OUT), jnp.float32),
            pltpu.VMEM((N_DEV - 1, CHUNK, D_OUT), jnp.float32),
            pltpu.VMEM((N_DEV - 1, CHUNK, D_OUT), jnp.float32),
            pltpu.SemaphoreType.DMA((N_DEV - 1,)),
            pltpu.SemaphoreType.DMA((N_DEV - 1,)),
        ],
        compiler_params=pltpu.CompilerParams(collective_id=0),
    )(x, router_W, route_idx, expert_W)


# device time: 19916 ns/iter; 2.3252x vs baseline; 2.3252x over previous
import jax
import jax.numpy as jnp
from jax import lax
from jax.experimental import pallas as pl
from jax.experimental.pallas import tpu as pltpu

N_DEV = 16
N_TOK = 512
D_IN = 256
D_OUT = 512
N_EXP = 32
CHUNK = N_TOK // N_DEV


def kernel(x, router_W, route_idx, expert_W):
    def body(x_ref, rw_ref, idx_ref, ew_ref, out_ref,
             p_ref, recv_buf, send_sems, recv_sems):
        my = lax.axis_index("i")

        barrier_sem = pltpu.get_barrier_semaphore()
        for o in range(1, N_DEV):
            pl.semaphore_signal(
                barrier_sem, inc=1,
                device_id=(lax.rem(my + o, N_DEV),),
                device_id_type=pl.DeviceIdType.MESH,
            )

        xv = x_ref[:, :]
        scores = jnp.dot(xv, rw_ref[:, :], preferred_element_type=jnp.float32)
        s_max = jnp.max(scores, axis=-1, keepdims=True)
        pexp = jnp.exp(scores - s_max)
        probs = pexp / jnp.sum(pexp, axis=-1, keepdims=True)

        e_ids = lax.broadcasted_iota(jnp.int32, (N_TOK, N_EXP), 1)
        top_mask = (e_ids == idx_ref[:, 0:1]) | (e_ids == idx_ref[:, 1:2])
        gp = jnp.where(top_mask, probs, 0.0)
        gates = gp / jnp.sum(gp, axis=-1, keepdims=True)

        g0 = jnp.sum(jnp.where(e_ids == 2 * my, gates, 0.0),
                     axis=-1, keepdims=True)
        g1 = jnp.sum(jnp.where(e_ids == 2 * my + 1, gates, 0.0),
                     axis=-1, keepdims=True)

        p_ref[:, :] = (
            g0 * jnp.dot(xv, ew_ref[0], preferred_element_type=jnp.float32)
            + g1 * jnp.dot(xv, ew_ref[1], preferred_element_type=jnp.float32)
        )

        pl.semaphore_wait(barrier_sem, N_DEV - 1)

        sends = []
        for o in range(1, N_DEV):
            dst = lax.rem(my + o, N_DEV)
            rdma = pltpu.make_async_remote_copy(
                src_ref=p_ref.at[pl.ds(dst * CHUNK, CHUNK), :],
                dst_ref=recv_buf.at[o - 1],
                send_sem=send_sems.at[o - 1],
                recv_sem=recv_sems.at[o - 1],
                device_id=(dst,),
                device_id_type=pl.DeviceIdType.MESH,
            )
            rdma.start()
            sends.append(rdma)

        acc = p_ref[pl.ds(my * CHUNK, CHUNK), :]
        for s in range(N_DEV - 1):
            recv = pltpu.make_async_remote_copy(
                src_ref=recv_buf.at[s],
                dst_ref=recv_buf.at[s],
                send_sem=send_sems.at[s],
                recv_sem=recv_sems.at[s],
                device_id=(my,),
                device_id_type=pl.DeviceIdType.MESH,
            )
            recv.wait_recv()
            acc = acc + recv_buf[s]
        out_ref[:, :] = acc

        for rdma in sends:
            rdma.wait_send()

    return pl.pallas_call(
        body,
        out_shape=jax.ShapeDtypeStruct((CHUNK, D_OUT), jnp.float32),
        in_specs=[
            pl.BlockSpec(memory_space=pltpu.VMEM),
            pl.BlockSpec(memory_space=pltpu.VMEM),
            pl.BlockSpec(memory_space=pltpu.VMEM),
            pl.BlockSpec(memory_space=pltpu.VMEM),
        ],
        out_specs=pl.BlockSpec(memory_space=pltpu.VMEM),
        scratch_shapes=[
            pltpu.VMEM((N_TOK, D_OUT), jnp.float32),
            pltpu.VMEM((N_DEV - 1, CHUNK, D_OUT), jnp.float32),
            pltpu.SemaphoreType.DMA((N_DEV - 1,)),
            pltpu.SemaphoreType.DMA((N_DEV - 1,)),
        ],
        compiler_params=pltpu.CompilerParams(collective_id=0),
    )(x, router_W, route_idx, expert_W)
